# baseline (speedup 1.0000x reference)
import functools

import jax
import jax.numpy as jnp
from jax import lax
from jax.experimental import pallas as pl
from jax.experimental.pallas import tpu as pltpu
from jax.experimental.pallas import tpu_sc as plsc

D_MODEL = 64
SCALE = float(D_MODEL) ** 0.5

_V = 1000000
_B = 16384
_S = 50
_NC = 2
_NS = 16
_NW = _NC * _NS

_CP = pltpu.CompilerParams(use_tc_tiling_on_sc=True, needs_layout_passes=False)
_MESH = dict(core_axis_name="c", subcore_axis_name="s")

_IOTA16 = lambda: jax.lax.iota(jnp.int32, 16)


_NBLK = _V // 128
_NSB = _NBLK // 2
_SB_PER_W = -(-_NSB // (_NW * 4)) * 4


def _reformat_kernel(tt_hbm, tail_hbm, t2_hbm,
                     in0, in1, in2, in3, out0, out1,
                     si0, si1, si2, si3, so0, so1):
    wid = lax.axis_index("s") * _NC + lax.axis_index("c")
    base = wid * _SB_PER_W

    iota = _IOTA16()
    rowv = [(iota + 16 * g) % 64 for g in range(8)]
    colb = [jnp.full((16,), g // 4, jnp.int32) for g in range(8)]
    ins, sis = [in0, in1, in2, in3], [si0, si1, si2, si3]
    outs, sos = [out0, out1], [so0, so1]

    def in_src(sb):
        return tt_hbm.at[:, pl.ds(sb * 256, 256)]

    def out_dst(sb):
        return t2_hbm.at[pl.ds(sb * 128, 128), :]

    def sb_at(k, b):
        return jnp.minimum(base + 4 * k + b, _NSB - 1)

    for b in range(4):
        pltpu.async_copy(in_src(sb_at(0, b)), ins[b], sis[b])

    def grp_body(k, _):
        for b in range(4):
            ob = b % 2
            sb = sb_at(k, b)
            pltpu.make_async_copy(in_src(sb), ins[b], sis[b]).wait()
            if b < 2:
                @pl.when(k > 0)
                def _w():
                    pltpu.make_async_copy(outs[ob], out_dst(sb), sos[ob]).wait()
            else:
                pltpu.make_async_copy(outs[ob], out_dst(sb), sos[ob]).wait()

            def p_body(p, _):
                for g in range(8):
                    vals = plsc.load_gather(ins[b], [rowv[g], colb[g] + 2 * p])
                    outs[ob][p, pl.ds(16 * g, 16)] = vals * SCALE
                return 0

            lax.fori_loop(0, 128, p_body, 0, unroll=2)
            pltpu.async_copy(outs[ob], out_dst(sb), sos[ob])
            pltpu.async_copy(in_src(sb_at(k + 1, b)), ins[b], sis[b])
        return 0

    lax.fori_loop(0, _SB_PER_W // 4, grp_body, 0)

    last = _NSB - 1
    for b in range(4):
        pltpu.make_async_copy(in_src(last), ins[b], sis[b]).wait()
    for ob in range(2):
        pltpu.make_async_copy(outs[ob], out_dst(last), sos[ob]).wait()

    @pl.when(wid == _NW - 1)
    def _copy_tail():
        pltpu.sync_copy(tail_hbm, in0.at[pl.ds(0, 32), pl.ds(0, 128)])
        pltpu.sync_copy(in0.at[pl.ds(0, 32), pl.ds(0, 128)],
                        t2_hbm.at[pl.ds(_NBLK * 64, 32), :])


@jax.jit
def _reformat(tt, tail):
    fn = functools.partial(
        pl.kernel,
        mesh=plsc.VectorSubcoreMesh(**_MESH),
        out_type=jax.ShapeDtypeStruct((_V // 2, 128), jnp.float32),
        scratch_types=(
            [pltpu.VMEM((64, 256), jnp.float32) for _ in range(4)]
            + [pltpu.VMEM((128, 128), jnp.float32) for _ in range(2)]
            + [pltpu.SemaphoreType.DMA for _ in range(6)]
        ),
        compiler_params=_CP,
    )(_reformat_kernel)
    return fn(tt, tail)


_JT = _B // 128
_CHU = 16
_NCHUNK = _S * (_JT // _CHU)
_CH_PER_W = -(-_NCHUNK // _NW)


def _gather_kernel(xt_hbm, t2_hbm, out_hbm,
                   idxb, ix0, ix1, rows0, rows1, os0, os1,
                   sg0, sg1, so0, so1):
    wid = lax.axis_index("s") * _NC + lax.axis_index("c")

    iota = _IOTA16()
    jv = [iota + 16 * t for t in range(16)]
    ixs, rows, oss = [ix0, ix1], [rows0, rows1], [os0, os1]
    sgs, sos = [sg0, sg1], [so0, so1]

    def prep(u, b):
        for t in range(16):
            ixs[b][pl.ds(16 * t, 16)] = lax.shift_right_logical(
                idxb[pl.ds(u * 256 + 16 * t, 16)], 1)

    def extract_half(u, b, h):
        ov = [(idxb[pl.ds(u * 256 + 16 * t, 16)] & 1) * 64
              for t in range(8 * h, 8 * h + 8)]
        rb, ob = rows[b], oss[b]

        def c_body(c, _):
            for i, t in enumerate(range(8 * h, 8 * h + 8)):
                vals = plsc.load_gather(rb, [jv[t], ov[i] + c])
                ob[c, pl.ds(16 * t, 16)] = vals
            return 0

        lax.fori_loop(0, D_MODEL, c_body, 0, unroll=2)

    def chunk_body(m, _):
        cid = jnp.minimum(wid * _CH_PER_W + m, _NCHUNK - 1)
        s = cid // (_JT // _CHU)
        jt0 = (cid % (_JT // _CHU)) * _CHU

        pltpu.sync_copy(xt_hbm.at[s, pl.ds(jt0 * 128, _CHU * 128)], idxb)

        prep(0, 0)
        h_g = [pltpu.async_copy(t2_hbm.at[ix0], rows0, sg0), None]
        h_o = [None, None]
        for u in range(8):
            b = u % 2
            if u + 1 < 8:
                prep(u + 1, 1 - b)
                h_g[1 - b] = pltpu.async_copy(
                    t2_hbm.at[ixs[1 - b]], rows[1 - b], sgs[1 - b])
            h_g[b].wait()
            if h_o[b] is not None:
                h_o[b].wait()
            extract_half(u, b, 0)
            extract_half(u, b, 1)
            h_o[b] = pltpu.async_copy(
                oss[b],
                out_hbm.at[s, :, pl.ds((jt0 + 2 * u) * 128, 256)], sos[b])
        h_o[0].wait()
        h_o[1].wait()
        return 0

    lax.fori_loop(0, _CH_PER_W, chunk_body, 0)


@jax.jit
def _gather(xt, t2):
    fn = functools.partial(
        pl.kernel,
        mesh=plsc.VectorSubcoreMesh(**_MESH),
        out_type=jax.ShapeDtypeStruct((_S, D_MODEL, _B), jnp.float32),
        scratch_types=[
            pltpu.VMEM((_CHU * 128,), jnp.int32),
            pltpu.VMEM((256,), jnp.int32),
            pltpu.VMEM((256,), jnp.int32),
            pltpu.VMEM((256, 128), jnp.float32),
            pltpu.VMEM((256, 128), jnp.float32),
            pltpu.VMEM((D_MODEL, 256), jnp.float32),
            pltpu.VMEM((D_MODEL, 256), jnp.float32),
            pltpu.SemaphoreType.DMA,
            pltpu.SemaphoreType.DMA,
            pltpu.SemaphoreType.DMA,
            pltpu.SemaphoreType.DMA,
        ],
        compiler_params=_CP,
    )(_gather_kernel)
    return fn(xt, t2)


def kernel(x, table):
    xt = jnp.transpose(x)
    tt = jnp.transpose(table)
    tail = jnp.reshape(lax.slice(table, (_NBLK * 128, 0), (_V, D_MODEL)),
                       (32, 128)) * SCALE
    t2 = _reformat(tt, tail)
    out_p = _gather(xt, t2)
    return jnp.transpose(out_p, (2, 0, 1))


# speedup vs baseline: 1.8762x; 1.8762x over previous
import functools

import jax
import jax.numpy as jnp
from jax import lax
from jax.experimental import pallas as pl
from jax.experimental.pallas import tpu as pltpu
from jax.experimental.pallas import tpu_sc as plsc

D_MODEL = 64
SCALE = float(D_MODEL) ** 0.5

_V = 1000000
_B = 16384
_S = 50
_NC = 2
_NS = 16
_NW = _NC * _NS

_CP = pltpu.CompilerParams(use_tc_tiling_on_sc=True, needs_layout_passes=False)
_MESH = dict(core_axis_name="c", subcore_axis_name="s")

_IOTA16 = lambda: jax.lax.iota(jnp.int32, 16)


_NBLK = _V // 128
_NSB = _NBLK // 2
_SB_PER_W = -(-_NSB // (_NW * 4)) * 4


def _reformat_kernel(tt_hbm, tail_hbm, t2_hbm,
                     in0, in1, in2, in3, out0, out1,
                     si0, si1, si2, si3, so0, so1):
    wid = lax.axis_index("s") * _NC + lax.axis_index("c")
    base = wid * _SB_PER_W

    iota = _IOTA16()
    rowv = [(iota + 16 * g) % 64 for g in range(8)]
    colb = [jnp.full((16,), g // 4, jnp.int32) for g in range(8)]
    ins, sis = [in0, in1, in2, in3], [si0, si1, si2, si3]
    outs, sos = [out0, out1], [so0, so1]

    def in_src(sb):
        return tt_hbm.at[:, pl.ds(sb * 256, 256)]

    def out_dst(sb):
        return t2_hbm.at[pl.ds(sb * 128, 128), :]

    def sb_at(k, b):
        return jnp.minimum(base + 4 * k + b, _NSB - 1)

    for b in range(4):
        pltpu.async_copy(in_src(sb_at(0, b)), ins[b], sis[b])

    def grp_body(k, _):
        for b in range(4):
            ob = b % 2
            sb = sb_at(k, b)
            pltpu.make_async_copy(in_src(sb), ins[b], sis[b]).wait()
            if b < 2:
                @pl.when(k > 0)
                def _w():
                    pltpu.make_async_copy(outs[ob], out_dst(sb), sos[ob]).wait()
            else:
                pltpu.make_async_copy(outs[ob], out_dst(sb), sos[ob]).wait()

            in_b, out_b = ins[b], outs[ob]

            @plsc.parallel_loop(0, 128, step=1, unroll=4)
            def _t(p):
                for g in range(8):
                    vals = plsc.load_gather(in_b, [rowv[g], colb[g] + 2 * p])
                    out_b[p, pl.ds(16 * g, 16)] = vals * SCALE
            pltpu.async_copy(outs[ob], out_dst(sb), sos[ob])
            pltpu.async_copy(in_src(sb_at(k + 1, b)), ins[b], sis[b])
        return 0

    lax.fori_loop(0, _SB_PER_W // 4, grp_body, 0)

    last = _NSB - 1
    for b in range(4):
        pltpu.make_async_copy(in_src(last), ins[b], sis[b]).wait()
    for ob in range(2):
        pltpu.make_async_copy(outs[ob], out_dst(last), sos[ob]).wait()

    @pl.when(wid == _NW - 1)
    def _copy_tail():
        pltpu.sync_copy(tail_hbm, in0.at[pl.ds(0, 32), pl.ds(0, 128)])
        pltpu.sync_copy(in0.at[pl.ds(0, 32), pl.ds(0, 128)],
                        t2_hbm.at[pl.ds(_NBLK * 64, 32), :])


@jax.jit
def _reformat(tt, tail):
    fn = functools.partial(
        pl.kernel,
        mesh=plsc.VectorSubcoreMesh(**_MESH),
        out_type=jax.ShapeDtypeStruct((_V // 2, 128), jnp.float32),
        scratch_types=(
            [pltpu.VMEM((64, 256), jnp.float32) for _ in range(4)]
            + [pltpu.VMEM((128, 128), jnp.float32) for _ in range(2)]
            + [pltpu.SemaphoreType.DMA for _ in range(6)]
        ),
        compiler_params=_CP,
    )(_reformat_kernel)
    return fn(tt, tail)


_JT = _B // 128
_CHU = 16
_NCHUNK = _S * (_JT // _CHU)
_CH_PER_W = -(-_NCHUNK // _NW)


def _gather_kernel(xt_hbm, t2_hbm, out_hbm,
                   idxb, ix0, ix1, rows0, rows1, os0, os1,
                   sg0, sg1, so0, so1):
    wid = lax.axis_index("s") * _NC + lax.axis_index("c")

    iota = _IOTA16()
    jv = [iota + 16 * t for t in range(16)]
    ixs, rows, oss = [ix0, ix1], [rows0, rows1], [os0, os1]
    sgs, sos = [sg0, sg1], [so0, so1]

    def prep(u, b):
        for t in range(16):
            ixs[b][pl.ds(16 * t, 16)] = lax.shift_right_logical(
                idxb[pl.ds(u * 256 + 16 * t, 16)], 1)

    def extract_half(u, b, h):
        ov = [(idxb[pl.ds(u * 256 + 16 * t, 16)] & 1) * 64
              for t in range(8 * h, 8 * h + 8)]
        rb, ob = rows[b], oss[b]

        @plsc.parallel_loop(0, D_MODEL, step=1, unroll=4)
        def _e(c):
            for i, t in enumerate(range(8 * h, 8 * h + 8)):
                vals = plsc.load_gather(rb, [jv[t], ov[i] + c])
                ob[c, pl.ds(16 * t, 16)] = vals

    def chunk_body(m, _):
        cid = jnp.minimum(wid * _CH_PER_W + m, _NCHUNK - 1)
        s = cid // (_JT // _CHU)
        jt0 = (cid % (_JT // _CHU)) * _CHU

        pltpu.sync_copy(xt_hbm.at[s, pl.ds(jt0 * 128, _CHU * 128)], idxb)

        prep(0, 0)
        h_g = [pltpu.async_copy(t2_hbm.at[ix0], rows0, sg0), None]
        h_o = [None, None]
        for u in range(8):
            b = u % 2
            if u + 1 < 8:
                prep(u + 1, 1 - b)
                h_g[1 - b] = pltpu.async_copy(
                    t2_hbm.at[ixs[1 - b]], rows[1 - b], sgs[1 - b])
            h_g[b].wait()
            if h_o[b] is not None:
                h_o[b].wait()
            extract_half(u, b, 0)
            extract_half(u, b, 1)
            h_o[b] = pltpu.async_copy(
                oss[b],
                out_hbm.at[s, :, pl.ds((jt0 + 2 * u) * 128, 256)], sos[b])
        h_o[0].wait()
        h_o[1].wait()
        return 0

    lax.fori_loop(0, _CH_PER_W, chunk_body, 0)


@jax.jit
def _gather(xt, t2):
    fn = functools.partial(
        pl.kernel,
        mesh=plsc.VectorSubcoreMesh(**_MESH),
        out_type=jax.ShapeDtypeStruct((_S, D_MODEL, _B), jnp.float32),
        scratch_types=[
            pltpu.VMEM((_CHU * 128,), jnp.int32),
            pltpu.VMEM((256,), jnp.int32),
            pltpu.VMEM((256,), jnp.int32),
            pltpu.VMEM((256, 128), jnp.float32),
            pltpu.VMEM((256, 128), jnp.float32),
            pltpu.VMEM((D_MODEL, 256), jnp.float32),
            pltpu.VMEM((D_MODEL, 256), jnp.float32),
            pltpu.SemaphoreType.DMA,
            pltpu.SemaphoreType.DMA,
            pltpu.SemaphoreType.DMA,
            pltpu.SemaphoreType.DMA,
        ],
        compiler_params=_CP,
    )(_gather_kernel)
    return fn(xt, t2)


def kernel(x, table):
    xt = jnp.transpose(x)
    tt = jnp.transpose(table)
    tail = jnp.reshape(lax.slice(table, (_NBLK * 128, 0), (_V, D_MODEL)),
                       (32, 128)) * SCALE
    t2 = _reformat(tt, tail)
    out_p = _gather(xt, t2)
    return jnp.transpose(out_p, (2, 0, 1))


# speedup vs baseline: 2.6642x; 1.4199x over previous
import functools

import jax
import jax.numpy as jnp
from jax import lax
from jax.experimental import pallas as pl
from jax.experimental.pallas import tpu as pltpu
from jax.experimental.pallas import tpu_sc as plsc

D_MODEL = 64
SCALE = float(D_MODEL) ** 0.5

_V = 1000000
_B = 16384
_S = 50
_NC = 2
_NS = 16
_NW = _NC * _NS

_CP = pltpu.CompilerParams(use_tc_tiling_on_sc=True, needs_layout_passes=False)
_MESH = dict(core_axis_name="c", subcore_axis_name="s")

_IOTA16 = lambda: jax.lax.iota(jnp.int32, 16)


_NBLK = _V // 128
_NSB = _NBLK // 2
_SB_PER_W = -(-_NSB // (_NW * 4)) * 4


def _reformat_kernel(tt_hbm, tail_hbm, t2_hbm,
                     in0, in1, in2, in3, out0, out1,
                     si0, si1, si2, si3, so0, so1):
    wid = lax.axis_index("s") * _NC + lax.axis_index("c")
    base = wid * _SB_PER_W

    iota = _IOTA16()
    rowv = [(iota + 16 * g) % 64 for g in range(8)]
    colb = [jnp.full((16,), g // 4, jnp.int32) for g in range(8)]
    ins, sis = [in0, in1, in2, in3], [si0, si1, si2, si3]
    outs, sos = [out0, out1], [so0, so1]

    def in_src(sb):
        return tt_hbm.at[:, pl.ds(sb * 256, 256)]

    def out_dst(sb):
        return t2_hbm.at[pl.ds(sb * 128, 128), :]

    def sb_at(k, b):
        return jnp.minimum(base + 4 * k + b, _NSB - 1)

    for b in range(4):
        pltpu.async_copy(in_src(sb_at(0, b)), ins[b], sis[b])

    def grp_body(k, _):
        for b in range(4):
            ob = b % 2
            sb = sb_at(k, b)
            pltpu.make_async_copy(in_src(sb), ins[b], sis[b]).wait()
            if b < 2:
                @pl.when(k > 0)
                def _w():
                    pltpu.make_async_copy(outs[ob], out_dst(sb), sos[ob]).wait()
            else:
                pltpu.make_async_copy(outs[ob], out_dst(sb), sos[ob]).wait()

            in_b, out_b = ins[b], outs[ob]

            @plsc.parallel_loop(0, 128, step=1, unroll=4)
            def _t(p):
                for g in range(8):
                    vals = plsc.load_gather(in_b, [rowv[g], colb[g] + 2 * p])
                    out_b[p, pl.ds(16 * g, 16)] = vals * SCALE
            pltpu.async_copy(outs[ob], out_dst(sb), sos[ob])
            pltpu.async_copy(in_src(sb_at(k + 1, b)), ins[b], sis[b])
        return 0

    lax.fori_loop(0, _SB_PER_W // 4, grp_body, 0)

    last = _NSB - 1
    for b in range(4):
        pltpu.make_async_copy(in_src(last), ins[b], sis[b]).wait()
    for ob in range(2):
        pltpu.make_async_copy(outs[ob], out_dst(last), sos[ob]).wait()

    @pl.when(wid == _NW - 1)
    def _copy_tail():
        pltpu.sync_copy(tail_hbm, in0.at[pl.ds(0, 32), pl.ds(0, 128)])
        pltpu.sync_copy(in0.at[pl.ds(0, 32), pl.ds(0, 128)],
                        t2_hbm.at[pl.ds(_NBLK * 64, 32), :])


@jax.jit
def _reformat(tt, tail):
    fn = functools.partial(
        pl.kernel,
        mesh=plsc.VectorSubcoreMesh(**_MESH),
        out_type=jax.ShapeDtypeStruct((_V // 2, 128), jnp.float32),
        scratch_types=(
            [pltpu.VMEM((64, 256), jnp.float32) for _ in range(4)]
            + [pltpu.VMEM((128, 128), jnp.float32) for _ in range(2)]
            + [pltpu.SemaphoreType.DMA for _ in range(6)]
        ),
        compiler_params=_CP,
    )(_reformat_kernel)
    return fn(tt, tail)


_JT = _B // 128
_CHU = 16
_NCHUNK = _S * (_JT // _CHU)
_CH_PER_W = -(-_NCHUNK // _NW)


def _gather_kernel(xt_hbm, t2_hbm, out_hbm,
                   idxb, ix0, ix1, rows0, rows1, os0, os1,
                   sg0, sg1, so0, so1):
    wid = lax.axis_index("s") * _NC + lax.axis_index("c")

    iota = _IOTA16()
    jv = [iota + 16 * t for t in range(16)]
    ixs, rows, oss = [ix0, ix1], [rows0, rows1], [os0, os1]
    sgs, sos = [sg0, sg1], [so0, so1]

    def prep(u, b):
        for t in range(16):
            ixs[b][pl.ds(16 * t, 16)] = lax.shift_right_logical(
                idxb[pl.ds(u * 256 + 16 * t, 16)], 1)

    def extract_half(u, b, h):
        ov = [(idxb[pl.ds(u * 256 + 16 * t, 16)] & 1) * 64
              for t in range(8 * h, 8 * h + 8)]
        rb, ob = rows[b], oss[b]

        @plsc.parallel_loop(0, D_MODEL, step=1, unroll=4)
        def _e(c):
            cd = (c + iota) & (D_MODEL - 1)
            for i, t in enumerate(range(8 * h, 8 * h + 8)):
                vals = plsc.load_gather(rb, [jv[t], ov[i] + cd])
                plsc.store_scatter(ob, [cd, jv[t]], vals)

    def chunk_body(m, _):
        cid = jnp.minimum(wid * _CH_PER_W + m, _NCHUNK - 1)
        s = cid // (_JT // _CHU)
        jt0 = (cid % (_JT // _CHU)) * _CHU

        pltpu.sync_copy(xt_hbm.at[s, pl.ds(jt0 * 128, _CHU * 128)], idxb)

        prep(0, 0)
        h_g = [pltpu.async_copy(t2_hbm.at[ix0], rows0, sg0), None]
        h_o = [None, None]
        for u in range(8):
            b = u % 2
            if u + 1 < 8:
                prep(u + 1, 1 - b)
                h_g[1 - b] = pltpu.async_copy(
                    t2_hbm.at[ixs[1 - b]], rows[1 - b], sgs[1 - b])
            h_g[b].wait()
            if h_o[b] is not None:
                h_o[b].wait()
            extract_half(u, b, 0)
            extract_half(u, b, 1)
            h_o[b] = pltpu.async_copy(
                oss[b],
                out_hbm.at[s, :, pl.ds((jt0 + 2 * u) * 128, 256)], sos[b])
        h_o[0].wait()
        h_o[1].wait()
        return 0

    lax.fori_loop(0, _CH_PER_W, chunk_body, 0)


@jax.jit
def _gather(xt, t2):
    fn = functools.partial(
        pl.kernel,
        mesh=plsc.VectorSubcoreMesh(**_MESH),
        out_type=jax.ShapeDtypeStruct((_S, D_MODEL, _B), jnp.float32),
        scratch_types=[
            pltpu.VMEM((_CHU * 128,), jnp.int32),
            pltpu.VMEM((256,), jnp.int32),
            pltpu.VMEM((256,), jnp.int32),
            pltpu.VMEM((256, 128), jnp.float32),
            pltpu.VMEM((256, 128), jnp.float32),
            pltpu.VMEM((D_MODEL, 256), jnp.float32),
            pltpu.VMEM((D_MODEL, 256), jnp.float32),
            pltpu.SemaphoreType.DMA,
            pltpu.SemaphoreType.DMA,
            pltpu.SemaphoreType.DMA,
            pltpu.SemaphoreType.DMA,
        ],
        compiler_params=_CP,
    )(_gather_kernel)
    return fn(xt, t2)


def kernel(x, table):
    xt = jnp.transpose(x)
    tt = jnp.transpose(table)
    tail = jnp.reshape(lax.slice(table, (_NBLK * 128, 0), (_V, D_MODEL)),
                       (32, 128)) * SCALE
    t2 = _reformat(tt, tail)
    out_p = _gather(xt, t2)
    return jnp.transpose(out_p, (2, 0, 1))


# speedup vs baseline: 3.1023x; 1.1645x over previous
import functools

import jax
import jax.numpy as jnp
from jax import lax
from jax.experimental import pallas as pl
from jax.experimental.pallas import tpu as pltpu
from jax.experimental.pallas import tpu_sc as plsc

D_MODEL = 64
SCALE = float(D_MODEL) ** 0.5

_V = 1000000
_B = 16384
_S = 50
_NC = 2
_NS = 16
_NW = _NC * _NS

_CP = pltpu.CompilerParams(use_tc_tiling_on_sc=True, needs_layout_passes=False)
_MESH = dict(core_axis_name="c", subcore_axis_name="s")

_IOTA16 = lambda: jax.lax.iota(jnp.int32, 16)


_NBLK = _V // 128
_NSB = _NBLK // 2
_SB_PER_W = -(-_NSB // (_NW * 4)) * 4


def _reformat_kernel(tt_hbm, tail_hbm, t2_hbm,
                     in0, in1, out0, out1, si0, si1, so0, so1):
    wid = lax.axis_index("s") * _NC + lax.axis_index("c")
    base = wid * _SB_PER_W

    iota = _IOTA16()
    mv = [(iota + k) & 15 for k in range(16)]
    rowv4 = [iota + 16 * gm for gm in range(4)]
    ins, sis = [in0, in1], [si0, si1]
    outs, sos = [out0, out1], [so0, so1]

    def in_src(sb):
        return tt_hbm.at[:, pl.ds(sb * 256, 256)]

    def out_dst(sb):
        return t2_hbm.at[pl.ds(sb * 128, 128), :]

    def sb_at(k, b):
        return jnp.minimum(base + 2 * k + b, _NSB - 1)

    for b in range(2):
        pltpu.async_copy(in_src(sb_at(0, b)), ins[b], sis[b])

    def grp_body(k, _):
        for b in range(2):
            sb = sb_at(k, b)
            pltpu.make_async_copy(in_src(sb), ins[b], sis[b]).wait()

            @pl.when(k > 0)
            def _w():
                pltpu.make_async_copy(outs[b], out_dst(sb), sos[b]).wait()

            in_b, out_b = ins[b], outs[b]

            for kd in range(16):
                mvk = mv[kd]
                for gm in range(4):
                    rv = rowv4[gm]

                    @plsc.parallel_loop(0, 16, step=1, unroll=2)
                    def _t(hh):
                        p0 = (hh & 7) * 16
                        ghi = hh >> 3
                        colv = (mvk << 1) + (2 * p0 + ghi)
                        vals = plsc.load_gather(in_b, [rv, colv])
                        pv = p0 + mvk
                        qv = (16 * gm + 64 * ghi) + iota
                        plsc.store_scatter(out_b, [pv, qv], vals * SCALE)

            pltpu.async_copy(outs[b], out_dst(sb), sos[b])
            pltpu.async_copy(in_src(sb_at(k + 1, b)), ins[b], sis[b])
        return 0

    lax.fori_loop(0, _SB_PER_W // 2, grp_body, 0)

    last = _NSB - 1
    for b in range(2):
        pltpu.make_async_copy(in_src(last), ins[b], sis[b]).wait()
        pltpu.make_async_copy(outs[b], out_dst(last), sos[b]).wait()

    @pl.when(wid == _NW - 1)
    def _copy_tail():
        pltpu.sync_copy(tail_hbm, in0.at[pl.ds(0, 32), pl.ds(0, 128)])
        pltpu.sync_copy(in0.at[pl.ds(0, 32), pl.ds(0, 128)],
                        t2_hbm.at[pl.ds(_NBLK * 64, 32), :])


@jax.jit
def _reformat(tt, tail):
    fn = functools.partial(
        pl.kernel,
        mesh=plsc.VectorSubcoreMesh(**_MESH),
        out_type=jax.ShapeDtypeStruct((_V // 2, 128), jnp.float32),
        scratch_types=(
            [pltpu.VMEM((64, 256), jnp.float32) for _ in range(2)]
            + [pltpu.VMEM((128, 128), jnp.float32) for _ in range(2)]
            + [pltpu.SemaphoreType.DMA for _ in range(4)]
        ),
        compiler_params=_CP,
    )(_reformat_kernel)
    return fn(tt, tail)


_JT = _B // 128
_CHU = 16
_NCHUNK = _S * (_JT // _CHU)
_CH_PER_W = -(-_NCHUNK // _NW)


def _gather_kernel(xt_hbm, t2_hbm, out_hbm,
                   idxb, ix0, ix1, rows0, rows1, os0, os1,
                   sg0, sg1, so0, so1):
    wid = lax.axis_index("s") * _NC + lax.axis_index("c")

    iota = _IOTA16()
    jv = [iota + 16 * t for t in range(16)]
    ixs, rows, oss = [ix0, ix1], [rows0, rows1], [os0, os1]
    sgs, sos = [sg0, sg1], [so0, so1]

    def prep(u, b):
        for t in range(16):
            ixs[b][pl.ds(16 * t, 16)] = lax.shift_right_logical(
                idxb[pl.ds(u * 256 + 16 * t, 16)], 1)

    def extract_half(u, b, h):
        ov = [(idxb[pl.ds(u * 256 + 16 * t, 16)] & 1) * 64
              for t in range(8 * h, 8 * h + 8)]
        rb, ob = rows[b], oss[b]

        @plsc.parallel_loop(0, D_MODEL, step=1, unroll=4)
        def _e(c):
            cd = (c + iota) & (D_MODEL - 1)
            for i, t in enumerate(range(8 * h, 8 * h + 8)):
                vals = plsc.load_gather(rb, [jv[t], ov[i] + cd])
                plsc.store_scatter(ob, [cd, jv[t]], vals)

    def chunk_body(m, _):
        cid = jnp.minimum(wid * _CH_PER_W + m, _NCHUNK - 1)
        s = cid // (_JT // _CHU)
        jt0 = (cid % (_JT // _CHU)) * _CHU

        pltpu.sync_copy(xt_hbm.at[s, pl.ds(jt0 * 128, _CHU * 128)], idxb)

        prep(0, 0)
        h_g = [pltpu.async_copy(t2_hbm.at[ix0], rows0, sg0), None]
        h_o = [None, None]
        for u in range(8):
            b = u % 2
            if u + 1 < 8:
                prep(u + 1, 1 - b)
                h_g[1 - b] = pltpu.async_copy(
                    t2_hbm.at[ixs[1 - b]], rows[1 - b], sgs[1 - b])
            h_g[b].wait()
            if h_o[b] is not None:
                h_o[b].wait()
            extract_half(u, b, 0)
            extract_half(u, b, 1)
            h_o[b] = pltpu.async_copy(
                oss[b],
                out_hbm.at[s, :, pl.ds((jt0 + 2 * u) * 128, 256)], sos[b])
        h_o[0].wait()
        h_o[1].wait()
        return 0

    lax.fori_loop(0, _CH_PER_W, chunk_body, 0)


@jax.jit
def _gather(xt, t2):
    fn = functools.partial(
        pl.kernel,
        mesh=plsc.VectorSubcoreMesh(**_MESH),
        out_type=jax.ShapeDtypeStruct((_S, D_MODEL, _B), jnp.float32),
        scratch_types=[
            pltpu.VMEM((_CHU * 128,), jnp.int32),
            pltpu.VMEM((256,), jnp.int32),
            pltpu.VMEM((256,), jnp.int32),
            pltpu.VMEM((256, 128), jnp.float32),
            pltpu.VMEM((256, 128), jnp.float32),
            pltpu.VMEM((D_MODEL, 256), jnp.float32),
            pltpu.VMEM((D_MODEL, 256), jnp.float32),
            pltpu.SemaphoreType.DMA,
            pltpu.SemaphoreType.DMA,
            pltpu.SemaphoreType.DMA,
            pltpu.SemaphoreType.DMA,
        ],
        compiler_params=_CP,
    )(_gather_kernel)
    return fn(xt, t2)


def kernel(x, table):
    xt = jnp.transpose(x)
    tt = jnp.transpose(table)
    tail = jnp.reshape(lax.slice(table, (_NBLK * 128, 0), (_V, D_MODEL)),
                       (32, 128)) * SCALE
    t2 = _reformat(tt, tail)
    out_p = _gather(xt, t2)
    return jnp.transpose(out_p, (2, 0, 1))


# speedup vs baseline: 5.1949x; 1.6745x over previous
import functools

import jax
import jax.numpy as jnp
from jax import lax
from jax.experimental import pallas as pl
from jax.experimental.pallas import tpu as pltpu
from jax.experimental.pallas import tpu_sc as plsc

D_MODEL = 64
SCALE = float(D_MODEL) ** 0.5

_V = 1000000
_B = 16384
_S = 50
_NC = 2
_NS = 16
_NW = _NC * _NS

_CP = pltpu.CompilerParams(use_tc_tiling_on_sc=True, needs_layout_passes=False)
_MESH = dict(core_axis_name="c", subcore_axis_name="s")

_IOTA16 = lambda: jax.lax.iota(jnp.int32, 16)


_NBLK = _V // 128
_NSB = _NBLK // 2
_SB_PER_W = -(-_NSB // (_NW * 4)) * 4


def _reformat_kernel(tt_hbm, tail_hbm, t2_hbm,
                     in0, in1, in2, in3, out0, out1,
                     si0, si1, si2, si3, so0, so1):
    wid = lax.axis_index("s") * _NC + lax.axis_index("c")
    base = wid * _SB_PER_W

    iota = _IOTA16()
    ins, sis = [in0, in1, in2, in3], [si0, si1, si2, si3]
    outs, sos = [out0, out1], [so0, so1]

    def in_src(sb):
        return tt_hbm.at[:, pl.ds(sb * 256, 256)]

    def out_dst(sb):
        return t2_hbm.at[pl.ds(sb * 128, 128), :]

    def sb_at(k, b):
        return jnp.minimum(base + 4 * k + b, _NSB - 1)

    for b in range(4):
        pltpu.async_copy(in_src(sb_at(0, b)), ins[b], sis[b])

    def grp_body(k, _):
        for b in range(4):
            ob = b % 2
            sb = sb_at(k, b)
            pltpu.make_async_copy(in_src(sb), ins[b], sis[b]).wait()
            if b < 2:
                @pl.when(k > 0)
                def _w():
                    pltpu.make_async_copy(outs[ob], out_dst(sb), sos[ob]).wait()
            else:
                pltpu.make_async_copy(outs[ob], out_dst(sb), sos[ob]).wait()

            in_b, out_b = ins[b], outs[ob]

            @plsc.parallel_loop(0, 1024, step=1, unroll=4)
            def _t(f):
                kd = f & 15
                gm = (f >> 4) & 3
                p0 = ((f >> 6) & 7) * 16
                ghi = f >> 9
                mvk = (kd + iota) & 15
                rv = 16 * gm + iota
                colv = (mvk << 1) + (2 * p0 + ghi)
                vals = plsc.load_gather(in_b, [rv, colv])
                pv = p0 + mvk
                qv = (16 * gm + 64 * ghi) + iota
                plsc.store_scatter(out_b, [pv, qv], vals * SCALE)

            pltpu.async_copy(outs[ob], out_dst(sb), sos[ob])
            pltpu.async_copy(in_src(sb_at(k + 1, b)), ins[b], sis[b])
        return 0

    lax.fori_loop(0, _SB_PER_W // 4, grp_body, 0)

    last = _NSB - 1
    for b in range(4):
        pltpu.make_async_copy(in_src(last), ins[b], sis[b]).wait()
    for ob in range(2):
        pltpu.make_async_copy(outs[ob], out_dst(last), sos[ob]).wait()

    @pl.when(wid == _NW - 1)
    def _copy_tail():
        pltpu.sync_copy(tail_hbm, in0.at[pl.ds(0, 32), pl.ds(0, 128)])
        pltpu.sync_copy(in0.at[pl.ds(0, 32), pl.ds(0, 128)],
                        t2_hbm.at[pl.ds(_NBLK * 64, 32), :])


@jax.jit
def _reformat(tt, tail):
    fn = functools.partial(
        pl.kernel,
        mesh=plsc.VectorSubcoreMesh(**_MESH),
        out_type=jax.ShapeDtypeStruct((_V // 2, 128), jnp.float32),
        scratch_types=(
            [pltpu.VMEM((64, 256), jnp.float32) for _ in range(4)]
            + [pltpu.VMEM((128, 128), jnp.float32) for _ in range(2)]
            + [pltpu.SemaphoreType.DMA for _ in range(6)]
        ),
        compiler_params=_CP,
    )(_reformat_kernel)
    return fn(tt, tail)


_JT = _B // 128
_CHU = 16
_NCHUNK = _S * (_JT // _CHU)
_CH_PER_W = -(-_NCHUNK // _NW)


def _gather_kernel(xt_hbm, t2_hbm, out_hbm,
                   idxb, ix0, ix1, rows0, rows1, os0, os1,
                   sg0, sg1, so0, so1):
    wid = lax.axis_index("s") * _NC + lax.axis_index("c")

    iota = _IOTA16()
    jv = [iota + 16 * t for t in range(16)]
    ixs, rows, oss = [ix0, ix1], [rows0, rows1], [os0, os1]
    sgs, sos = [sg0, sg1], [so0, so1]

    def prep(u, b):
        for t in range(16):
            ixs[b][pl.ds(16 * t, 16)] = lax.shift_right_logical(
                idxb[pl.ds(u * 256 + 16 * t, 16)], 1)

    def extract_half(u, b, h):
        ov = [(idxb[pl.ds(u * 256 + 16 * t, 16)] & 1) * 64
              for t in range(8 * h, 8 * h + 8)]
        rb, ob = rows[b], oss[b]

        @plsc.parallel_loop(0, D_MODEL, step=1, unroll=4)
        def _e(c):
            cd = (c + iota) & (D_MODEL - 1)
            for i, t in enumerate(range(8 * h, 8 * h + 8)):
                vals = plsc.load_gather(rb, [jv[t], ov[i] + cd])
                plsc.store_scatter(ob, [cd, jv[t]], vals)

    def chunk_body(m, _):
        cid = jnp.minimum(wid * _CH_PER_W + m, _NCHUNK - 1)
        s = cid // (_JT // _CHU)
        jt0 = (cid % (_JT // _CHU)) * _CHU

        pltpu.sync_copy(xt_hbm.at[s, pl.ds(jt0 * 128, _CHU * 128)], idxb)

        prep(0, 0)
        h_g = [pltpu.async_copy(t2_hbm.at[ix0], rows0, sg0), None]
        h_o = [None, None]
        for u in range(8):
            b = u % 2
            if u + 1 < 8:
                prep(u + 1, 1 - b)
                h_g[1 - b] = pltpu.async_copy(
                    t2_hbm.at[ixs[1 - b]], rows[1 - b], sgs[1 - b])
            h_g[b].wait()
            if h_o[b] is not None:
                h_o[b].wait()
            extract_half(u, b, 0)
            extract_half(u, b, 1)
            h_o[b] = pltpu.async_copy(
                oss[b],
                out_hbm.at[s, :, pl.ds((jt0 + 2 * u) * 128, 256)], sos[b])
        h_o[0].wait()
        h_o[1].wait()
        return 0

    lax.fori_loop(0, _CH_PER_W, chunk_body, 0)


@jax.jit
def _gather(xt, t2):
    fn = functools.partial(
        pl.kernel,
        mesh=plsc.VectorSubcoreMesh(**_MESH),
        out_type=jax.ShapeDtypeStruct((_S, D_MODEL, _B), jnp.float32),
        scratch_types=[
            pltpu.VMEM((_CHU * 128,), jnp.int32),
            pltpu.VMEM((256,), jnp.int32),
            pltpu.VMEM((256,), jnp.int32),
            pltpu.VMEM((256, 128), jnp.float32),
            pltpu.VMEM((256, 128), jnp.float32),
            pltpu.VMEM((D_MODEL, 256), jnp.float32),
            pltpu.VMEM((D_MODEL, 256), jnp.float32),
            pltpu.SemaphoreType.DMA,
            pltpu.SemaphoreType.DMA,
            pltpu.SemaphoreType.DMA,
            pltpu.SemaphoreType.DMA,
        ],
        compiler_params=_CP,
    )(_gather_kernel)
    return fn(xt, t2)


def kernel(x, table):
    xt = jnp.transpose(x)
    tt = jnp.transpose(table)
    tail = jnp.reshape(lax.slice(table, (_NBLK * 128, 0), (_V, D_MODEL)),
                       (32, 128)) * SCALE
    t2 = _reformat(tt, tail)
    out_p = _gather(xt, t2)
    return jnp.transpose(out_p, (2, 0, 1))


# speedup vs baseline: 5.3440x; 1.0287x over previous
import functools

import jax
import jax.numpy as jnp
from jax import lax
from jax.experimental import pallas as pl
from jax.experimental.pallas import tpu as pltpu
from jax.experimental.pallas import tpu_sc as plsc

D_MODEL = 64
SCALE = float(D_MODEL) ** 0.5

_V = 1000000
_B = 16384
_S = 50
_NC = 2
_NS = 16
_NW = _NC * _NS

_CP = pltpu.CompilerParams(use_tc_tiling_on_sc=True, needs_layout_passes=False)
_MESH = dict(core_axis_name="c", subcore_axis_name="s")

_IOTA16 = lambda: jax.lax.iota(jnp.int32, 16)


_NBLK = _V // 128
_NSB = _NBLK // 2
_SB_PER_W = -(-_NSB // (_NW * 4)) * 4


def _reformat_kernel(tt_hbm, tail_hbm, t2_hbm,
                     in0, in1, in2, in3, out0, out1,
                     si0, si1, si2, si3, so0, so1):
    wid = lax.axis_index("s") * _NC + lax.axis_index("c")
    base = wid * _SB_PER_W

    iota = _IOTA16()
    ins, sis = [in0, in1, in2, in3], [si0, si1, si2, si3]
    outs, sos = [out0, out1], [so0, so1]

    def in_src(sb):
        return tt_hbm.at[:, pl.ds(sb * 256, 256)]

    def out_dst(sb):
        return t2_hbm.at[pl.ds(sb * 128, 128), :]

    def sb_at(k, b):
        return jnp.minimum(base + 4 * k + b, _NSB - 1)

    for b in range(4):
        pltpu.async_copy(in_src(sb_at(0, b)), ins[b], sis[b])

    def grp_body(k, _):
        for b in range(4):
            ob = b % 2
            sb = sb_at(k, b)
            pltpu.make_async_copy(in_src(sb), ins[b], sis[b]).wait()
            if b < 2:
                @pl.when(k > 0)
                def _w():
                    pltpu.make_async_copy(outs[ob], out_dst(sb), sos[ob]).wait()
            else:
                pltpu.make_async_copy(outs[ob], out_dst(sb), sos[ob]).wait()

            in_b, out_b = ins[b], outs[ob]

            @plsc.parallel_loop(0, 1024, step=1, unroll=4)
            def _t(f):
                kd = f & 15
                gm = (f >> 4) & 3
                p0 = ((f >> 6) & 7) * 16
                ghi = f >> 9
                mvk = (kd + iota) & 15
                rv = 16 * gm + iota
                colv = (mvk << 1) + (2 * p0 + ghi)
                vals = plsc.load_gather(in_b, [rv, colv])
                pv = p0 + mvk
                qv = (16 * gm + 64 * ghi) + iota
                plsc.store_scatter(out_b, [pv, qv], vals * SCALE)

            pltpu.async_copy(outs[ob], out_dst(sb), sos[ob])
            pltpu.async_copy(in_src(sb_at(k + 1, b)), ins[b], sis[b])
        return 0

    lax.fori_loop(0, _SB_PER_W // 4, grp_body, 0)

    last = _NSB - 1
    for b in range(4):
        pltpu.make_async_copy(in_src(last), ins[b], sis[b]).wait()
    for ob in range(2):
        pltpu.make_async_copy(outs[ob], out_dst(last), sos[ob]).wait()

    @pl.when(wid == _NW - 1)
    def _copy_tail():
        pltpu.sync_copy(tail_hbm, in0.at[pl.ds(0, 32), pl.ds(0, 128)])
        pltpu.sync_copy(in0.at[pl.ds(0, 32), pl.ds(0, 128)],
                        t2_hbm.at[pl.ds(_NBLK * 64, 32), :])


@jax.jit
def _reformat(tt, tail):
    fn = functools.partial(
        pl.kernel,
        mesh=plsc.VectorSubcoreMesh(**_MESH),
        out_type=jax.ShapeDtypeStruct((_V // 2, 128), jnp.float32),
        scratch_types=(
            [pltpu.VMEM((64, 256), jnp.float32) for _ in range(4)]
            + [pltpu.VMEM((128, 128), jnp.float32) for _ in range(2)]
            + [pltpu.SemaphoreType.DMA for _ in range(6)]
        ),
        compiler_params=_CP,
    )(_reformat_kernel)
    return fn(tt, tail)


_JT = _B // 128
_CHU = 16
_NCHUNK = _S * (_JT // _CHU)
_CH_PER_W = -(-_NCHUNK // _NW)


def _gather_kernel(xt_hbm, t2_hbm, out_hbm,
                   idxb, ix0, ix1, ix2, ix3, rows0, rows1, rows2, rows3,
                   os0, os1, sg0, sg1, sg2, sg3, so0, so1):
    wid = lax.axis_index("s") * _NC + lax.axis_index("c")

    iota = _IOTA16()
    jv = [iota + 16 * t for t in range(8)]
    ixs, rows = [ix0, ix1, ix2, ix3], [rows0, rows1, rows2, rows3]
    oss = [os0, os1]
    sgs, sos = [sg0, sg1, sg2, sg3], [so0, so1]

    def prep(u, b):
        for t in range(8):
            ixs[b][pl.ds(16 * t, 16)] = lax.shift_right_logical(
                idxb[pl.ds(u * 128 + 16 * t, 16)], 1)

    def extract(u, b4, b2):
        ov = [(idxb[pl.ds(u * 128 + 16 * t, 16)] & 1) * 64 for t in range(8)]
        rb, ob = rows[b4], oss[b2]

        @plsc.parallel_loop(0, D_MODEL, step=1, unroll=4)
        def _e(c):
            cd = (c + iota) & (D_MODEL - 1)
            for t in range(8):
                vals = plsc.load_gather(rb, [jv[t], ov[t] + cd])
                plsc.store_scatter(ob, [cd, jv[t]], vals)

    def chunk_body(m, _):
        cid = jnp.minimum(wid * _CH_PER_W + m, _NCHUNK - 1)
        s = cid // (_JT // _CHU)
        jt0 = (cid % (_JT // _CHU)) * _CHU

        pltpu.sync_copy(xt_hbm.at[s, pl.ds(jt0 * 128, _CHU * 128)], idxb)

        h_g = [None, None, None, None]
        h_o = [None, None]
        for u in range(3):
            prep(u, u)
            h_g[u] = pltpu.async_copy(t2_hbm.at[ixs[u]], rows[u], sgs[u])
        for u in range(16):
            b4, b2 = u % 4, u % 2
            if u + 3 < 16:
                nb = (u + 3) % 4
                prep(u + 3, nb)
                h_g[nb] = pltpu.async_copy(
                    t2_hbm.at[ixs[nb]], rows[nb], sgs[nb])
            h_g[b4].wait()
            if h_o[b2] is not None:
                h_o[b2].wait()
            extract(u, b4, b2)
            h_o[b2] = pltpu.async_copy(
                oss[b2],
                out_hbm.at[s, :, pl.ds((jt0 + u) * 128, 128)], sos[b2])
        h_o[0].wait()
        h_o[1].wait()
        return 0

    lax.fori_loop(0, _CH_PER_W, chunk_body, 0)


@jax.jit
def _gather(xt, t2):
    fn = functools.partial(
        pl.kernel,
        mesh=plsc.VectorSubcoreMesh(**_MESH),
        out_type=jax.ShapeDtypeStruct((_S, D_MODEL, _B), jnp.float32),
        scratch_types=(
            [pltpu.VMEM((_CHU * 128,), jnp.int32)]
            + [pltpu.VMEM((128,), jnp.int32) for _ in range(4)]
            + [pltpu.VMEM((128, 128), jnp.float32) for _ in range(4)]
            + [pltpu.VMEM((D_MODEL, 128), jnp.float32) for _ in range(2)]
            + [pltpu.SemaphoreType.DMA for _ in range(6)]
        ),
        compiler_params=_CP,
    )(_gather_kernel)
    return fn(xt, t2)


def kernel(x, table):
    xt = jnp.transpose(x)
    tt = jnp.transpose(table)
    tail = jnp.reshape(lax.slice(table, (_NBLK * 128, 0), (_V, D_MODEL)),
                       (32, 128)) * SCALE
    t2 = _reformat(tt, tail)
    out_p = _gather(xt, t2)
    return jnp.transpose(out_p, (2, 0, 1))
